# Initial kernel scaffold; baseline (speedup 1.0000x reference)
import jax
import jax.numpy as jnp
from jax.experimental import pallas as pl


def kernel(poses, bars, pos_table, bar_table):
    raise NotImplementedError("write your pallas kernel here")


# speedup vs baseline: 8.1825x; 8.1825x over previous
import math

import jax
import jax.numpy as jnp
from jax import lax
from jax.experimental import pallas as pl
from jax.experimental.pallas import tpu as pltpu
from jax.experimental.pallas import tpu_sc as plsc

D_EMB = 128
N_BAR_STEPS = 64
MAX_BAR = 10

NC = 2
NS = 16
NW = NC * NS
LANES = 16

N_TOKENS = 1024 * 200
PER_W = N_TOKENS // NW
CHUNK = 128
N_CHUNKS = PER_W // CHUNK
NBUF = 2


def _sc_lookup(table, poses1d, bars1d):
  mesh = plsc.VectorSubcoreMesh(core_axis_name="c", subcore_axis_name="s")

  def body(table_hbm, poses_hbm, bars_hbm, out_hbm, idx_v, bar_v, rows_v,
           gsem):
    wid = lax.axis_index("s") * NC + lax.axis_index("c")
    base = pl.multiple_of(wid * PER_W, PER_W)

    pltpu.sync_copy(poses_hbm.at[pl.ds(base, PER_W)], idx_v)
    pltpu.sync_copy(bars_hbm.at[pl.ds(base, PER_W)], bar_v)

    def fuse(i, _):
      s = pl.multiple_of(i * LANES, LANES)
      idx_v[pl.ds(s, LANES)] = (
          idx_v[pl.ds(s, LANES)] * MAX_BAR + bar_v[pl.ds(s, LANES)]
      )
      return ()
    lax.fori_loop(0, PER_W // LANES, fuse, (), unroll=8)

    def gather(j, buf):
      o = pl.multiple_of(j * CHUNK, CHUNK)
      return pltpu.async_copy(
          table_hbm.at[idx_v.at[pl.ds(o, CHUNK)]], rows_v.at[buf], gsem)

    gather(0, 0).wait()

    def step(j, _):
      buf = j % NBUF
      nxt = (j + 1) % NBUF

      @pl.when(j + 1 < N_CHUNKS)
      def _():
        gather(j + 1, nxt)

      o = pl.multiple_of((wid * N_CHUNKS + j) * CHUNK, CHUNK)
      pltpu.sync_copy(rows_v.at[buf], out_hbm.at[pl.ds(o, CHUNK)])

      @pl.when(j + 1 < N_CHUNKS)
      def _():
        oo = pl.multiple_of((j + 1) * CHUNK, CHUNK)
        pltpu.make_async_copy(
            table_hbm.at[idx_v.at[pl.ds(oo, CHUNK)]], rows_v.at[nxt], gsem
        ).wait()
      return ()

    lax.fori_loop(0, N_CHUNKS, step, ())

  return pl.kernel(
      body,
      out_type=jax.ShapeDtypeStruct((N_TOKENS, D_EMB), jnp.float32),
      mesh=mesh,
      scratch_types=[
          pltpu.VMEM((PER_W,), jnp.int32),
          pltpu.VMEM((PER_W,), jnp.int32),
          pltpu.VMEM((NBUF, CHUNK, D_EMB), jnp.float32),
          pltpu.SemaphoreType.DMA,
      ],
  )(table, poses1d, bars1d)


def kernel(poses, bars, pos_table, bar_table):
  scale = math.sqrt(D_EMB)
  fused = ((pos_table[:, None, :] + bar_table[None, :, :]) * scale).reshape(
      N_BAR_STEPS * MAX_BAR, D_EMB)
  out = _sc_lookup(fused, poses.reshape(-1), bars.reshape(-1))
  return out.reshape(poses.shape + (D_EMB,))
